# baseline (speedup 1.0000x reference)
import functools

import jax
import jax.numpy as jnp
from jax import lax
from jax.experimental import pallas as pl
from jax.experimental.pallas import tpu as pltpu
from jax.experimental.pallas import tpu_sc as plsc

M = 1024
DH = 2048
E = 8
T = 4096
K = 2

BLK = 256
NB = (K * T) // BLK + E
P_PAD = NB * BLK

NW = 32

_SQRT_HALF = 0.7071067811865476


def _gating_body(x_ref, wg_ref, wgb_ref, wn_ref, wnb_ref, noise_ref,
                 idx_ref, gv_ref, imp_ref, load_ref, xbf_ref):
    t = pl.program_id(0)
    x = x_ref[...]
    xbf = x.astype(jnp.bfloat16)
    lo = lax.bitcast_convert_type(xbf[:, :M // 2], jnp.uint16)
    hi = lax.bitcast_convert_type(xbf[:, M // 2:], jnp.uint16)
    packed = lo.astype(jnp.uint32) | (hi.astype(jnp.uint32) << 16)
    xbf_ref[...] = lax.bitcast_convert_type(packed, jnp.int32)
    logits = lax.dot_general(
        x, wg_ref[...], (((1,), (1,)), ((), ())),
        preferred_element_type=jnp.float32) + wgb_ref[...]
    pre = lax.dot_general(
        x, wn_ref[...], (((1,), (1,)), ((), ())),
        preferred_element_type=jnp.float32) + wnb_ref[...]
    noise_scale = jax.nn.softplus(pre)
    h = logits + noise_ref[...] * noise_scale

    tm = h.shape[0]
    iota = lax.broadcasted_iota(jnp.int32, (tm, E), 1)
    neg_inf = jnp.float32(-jnp.inf)

    m1 = jnp.max(h, axis=1, keepdims=True)
    i1 = jnp.min(jnp.where(h == m1, iota, E), axis=1, keepdims=True)
    mask1 = iota == i1
    h2 = jnp.where(mask1, neg_inf, h)
    m2 = jnp.max(h2, axis=1, keepdims=True)
    i2 = jnp.min(jnp.where(h2 == m2, iota, E), axis=1, keepdims=True)
    mask2 = iota == i2
    h3 = jnp.where(mask2, neg_inf, h2)
    m3 = jnp.max(h3, axis=1, keepdims=True)

    e2 = jnp.exp(m2 - m1)
    denom = 1.0 + e2
    g1 = 1.0 / denom
    g2 = e2 / denom
    gates = jnp.where(mask1, g1, 0.0) + jnp.where(mask2, g2, 0.0)
    idx_ref[...] = jnp.concatenate([i1, i2], axis=1)
    gv_ref[...] = jnp.concatenate([g1, g2], axis=1)

    psi = jnp.where(h > m2, m2, jnp.where(h <= m3, m3, h))
    z = (logits - psi) / noise_scale
    p = 0.5 * (1.0 + lax.erf(z * _SQRT_HALF))

    imp_part = jnp.sum(gates, axis=0, keepdims=True)
    load_part = jnp.sum(p, axis=0, keepdims=True)

    @pl.when(t == 0)
    def _():
        imp_ref[...] = imp_part
        load_ref[...] = load_part

    @pl.when(t != 0)
    def _():
        imp_ref[...] += imp_part
        load_ref[...] += load_part


def _gating(flat, wg_w, wg_b, wn_w, wn_b, noise, tm=1024):
    nt = T // tm
    return pl.pallas_call(
        _gating_body,
        grid=(nt,),
        in_specs=[
            pl.BlockSpec((tm, M), lambda t: (t, 0)),
            pl.BlockSpec((E, M), lambda t: (0, 0)),
            pl.BlockSpec((1, E), lambda t: (0, 0)),
            pl.BlockSpec((E, M), lambda t: (0, 0)),
            pl.BlockSpec((1, E), lambda t: (0, 0)),
            pl.BlockSpec((tm, E), lambda t: (t, 0)),
        ],
        out_specs=[
            pl.BlockSpec((tm, 2), lambda t: (t, 0)),
            pl.BlockSpec((tm, 2), lambda t: (t, 0)),
            pl.BlockSpec((1, E), lambda t: (0, 0)),
            pl.BlockSpec((1, E), lambda t: (0, 0)),
            pl.BlockSpec((tm, M // 2), lambda t: (t, 0)),
        ],
        out_shape=[
            jax.ShapeDtypeStruct((T, 2), jnp.int32),
            jax.ShapeDtypeStruct((T, 2), jnp.float32),
            jax.ShapeDtypeStruct((1, E), jnp.float32),
            jax.ShapeDtypeStruct((1, E), jnp.float32),
            jax.ShapeDtypeStruct((T, M // 2), jnp.int32),
        ],
    )(flat, wg_w, wg_b, wn_w, wn_b, noise)


def _route(idx, gv):
    a = jnp.concatenate([idx[:, 0], idx[:, 1]])
    g = jnp.concatenate([gv[:, 0], gv[:, 1]])
    ar = jnp.arange(T, dtype=jnp.int32)
    tok = jnp.concatenate([ar, ar])
    onehot = (a[:, None] == jnp.arange(E, dtype=jnp.int32)[None, :])
    onehot = onehot.astype(jnp.int32)
    csum = jnp.cumsum(onehot, axis=0)
    counts = csum[-1]
    rank = jnp.sum(csum * onehot, axis=1) - 1
    cnt_pad = ((counts + BLK - 1) // BLK) * BLK
    ends = jnp.cumsum(cnt_pad)
    offs = ends - cnt_pad
    pos = jnp.sum(offs[None, :] * onehot, axis=1) + rank
    tok_of_slot = jnp.zeros((P_PAD,), jnp.int32).at[pos].set(tok)
    gate_of_slot = jnp.zeros((P_PAD,), jnp.float32).at[pos].set(g)
    bstart = jnp.arange(NB, dtype=jnp.int32) * BLK
    block_expert = jnp.minimum(
        jnp.sum((bstart[:, None] >= ends[None, :]).astype(jnp.int32), axis=1),
        E - 1).astype(jnp.int32)
    return tok_of_slot, gate_of_slot, block_expert, pos[:T], pos[T:]


def _sc_gather(flat_pk, tok_of_slot):
    m2 = M // 2
    n_slots = tok_of_slot.shape[0]
    rows_per_w = n_slots // NW
    c = 80
    nch = rows_per_w // c
    assert rows_per_w % c == 0
    mesh = plsc.VectorSubcoreMesh(core_axis_name="c", subcore_axis_name="s")

    @functools.partial(
        pl.kernel, mesh=mesh,
        out_type=jax.ShapeDtypeStruct((n_slots, m2), jnp.int32),
        scratch_types=[
            pltpu.VMEM((rows_per_w,), jnp.int32),
            pltpu.VMEM((c, m2), jnp.int32),
            pltpu.VMEM((c, m2), jnp.int32),
            pltpu.SemaphoreType.DMA,
            pltpu.SemaphoreType.DMA,
        ],
    )
    def k(flat_hbm, idx_hbm, out_hbm, idx_v, buf_a, buf_b, sem_a, sem_b):
        wid = lax.axis_index("s") * 2 + lax.axis_index("c")
        base = wid * rows_per_w
        pltpu.sync_copy(idx_hbm.at[pl.ds(base, rows_per_w)], idx_v)
        bufs = (buf_a, buf_b)
        sems = (sem_a, sem_b)
        cps = [None, None]
        for ch in range(nch):
            p = ch % 2
            cps[p] = pltpu.async_copy(
                flat_hbm.at[idx_v.at[pl.ds(ch * c, c)]], bufs[p], sems[p])
            if ch >= 1:
                q = (ch - 1) % 2
                cps[q].wait()
                pltpu.sync_copy(bufs[q],
                                out_hbm.at[pl.ds(base + (ch - 1) * c, c)])
        cps[(nch - 1) % 2].wait()
        pltpu.sync_copy(bufs[(nch - 1) % 2],
                        out_hbm.at[pl.ds(base + (nch - 1) * c, c)])

    return k(flat_pk, tok_of_slot)


def _expert_body(be_ref, x_ref, g_ref, ww_ref, wb_ref, vw_ref, vb_ref,
                 w2w_ref, w2b_ref, out_ref):
    u = lax.bitcast_convert_type(x_ref[...], jnp.uint32)
    lo = lax.bitcast_convert_type((u & 0xFFFF).astype(jnp.uint16),
                                  jnp.bfloat16)
    hi = lax.bitcast_convert_type((u >> 16).astype(jnp.uint16),
                                  jnp.bfloat16)
    x = jnp.concatenate([lo, hi], axis=1)
    xv = lax.dot_general(
        x, vw_ref[...], (((1,), (1,)), ((), ())),
        preferred_element_type=jnp.float32) + vb_ref[...]
    xw = lax.dot_general(
        x, ww_ref[0], (((1,), (1,)), ((), ())),
        preferred_element_type=jnp.float32) + wb_ref[0]
    act = xw * jax.nn.sigmoid(xw) * xv
    contrib = lax.dot_general(
        act.astype(jnp.bfloat16), w2w_ref[...], (((1,), (1,)), ((), ())),
        preferred_element_type=jnp.float32) + w2b_ref[...]
    g = g_ref[0, 0, :].reshape(-1, 1)
    out_ref[...] = g * contrib


def _experts(block_expert, x_sorted, gate_of_slot, ww, wb, vw, vb, w2w, w2b):
    grid_spec = pltpu.PrefetchScalarGridSpec(
        num_scalar_prefetch=1,
        grid=(NB,),
        in_specs=[
            pl.BlockSpec((BLK, M // 2), lambda b, be: (b, 0)),
            pl.BlockSpec((1, 1, BLK), lambda b, be: (b, 0, 0)),
            pl.BlockSpec((1, DH, M), lambda b, be: (be[b], 0, 0)),
            pl.BlockSpec((1, 1, DH), lambda b, be: (be[b], 0, 0)),
            pl.BlockSpec((DH, M), lambda b, be: (0, 0)),
            pl.BlockSpec((1, DH), lambda b, be: (0, 0)),
            pl.BlockSpec((M, DH), lambda b, be: (0, 0)),
            pl.BlockSpec((1, M), lambda b, be: (0, 0)),
        ],
        out_specs=pl.BlockSpec((BLK, M), lambda b, be: (b, 0)),
    )
    return pl.pallas_call(
        _expert_body,
        grid_spec=grid_spec,
        out_shape=jax.ShapeDtypeStruct((P_PAD, M), jnp.float32),
    )(block_expert, x_sorted, gate_of_slot.reshape(NB, 1, BLK), ww, wb,
      vw, vb, w2w, w2b)


def _sc_combine(rows, slot1, slot2):
    rows_per_w = T // NW
    c = 32
    nch = rows_per_w // c
    nsl = M // 16
    mesh = plsc.VectorSubcoreMesh(core_axis_name="c", subcore_axis_name="s")

    @functools.partial(
        pl.kernel, mesh=mesh,
        out_type=jax.ShapeDtypeStruct((T, M), jnp.float32),
        scratch_types=[
            pltpu.VMEM((c,), jnp.int32),
            pltpu.VMEM((c,), jnp.int32),
            pltpu.VMEM((c, M), jnp.float32),
            pltpu.VMEM((c, M), jnp.float32),
            pltpu.SemaphoreType.DMA,
            pltpu.SemaphoreType.DMA,
        ],
    )
    def k(rows_hbm, s1_hbm, s2_hbm, out_hbm, i1_v, i2_v, a_v, b_v, sa, sb):
        wid = lax.axis_index("s") * 2 + lax.axis_index("c")
        base = wid * rows_per_w

        def body(i, carry):
            b = base + i * c
            pltpu.sync_copy(s1_hbm.at[pl.ds(b, c)], i1_v)
            pltpu.sync_copy(s2_hbm.at[pl.ds(b, c)], i2_v)
            cp1 = pltpu.async_copy(rows_hbm.at[i1_v], a_v, sa)
            cp2 = pltpu.async_copy(rows_hbm.at[i2_v], b_v, sb)
            cp1.wait()
            cp2.wait()

            def add_row(j, carry2):
                for s in range(nsl):
                    sl = pl.ds(s * 16, 16)
                    a_v[j, sl] = a_v[j, sl] + b_v[j, sl]
                return carry2

            lax.fori_loop(0, c, add_row, 0)
            pltpu.sync_copy(a_v, out_hbm.at[pl.ds(b, c)])
            return carry

        lax.fori_loop(0, nch, body, 0)

    return k(rows, slot1, slot2)


def _cv_loss(v):
    return 0.01 * jnp.std(v) / (jnp.mean(v) + 1e-6)


def kernel(x, Wg_w, Wg_b, Wn_w, Wn_b, W_w, W_b, V_w, V_b, W2_w, W2_b):
    B, N, m = x.shape
    flat = x.reshape(B * N, m)
    noise = jax.random.normal(jax.random.key(1234), (T, E),
                              dtype=jnp.float32)

    idx, gv, imp, load, flat_pk = _gating(
        flat, Wg_w, Wg_b.reshape(1, E), Wn_w, Wn_b.reshape(1, E), noise)
    l_moe = _cv_loss(imp[0]) + _cv_loss(load[0])

    tok_of_slot, gate_of_slot, block_expert, slot1, slot2 = _route(idx, gv)

    x_sorted = _sc_gather(flat_pk, tok_of_slot)

    rows = _experts(
        block_expert, x_sorted, gate_of_slot,
        W_w.astype(jnp.bfloat16), W_b.reshape(E, 1, DH),
        V_w.astype(jnp.bfloat16), V_b.reshape(1, DH),
        W2_w.astype(jnp.bfloat16), W2_b.reshape(1, M))

    flat_out = _sc_combine(rows, slot1, slot2)
    return (flat_out.reshape(B, N, m), l_moe)


# speedup vs baseline: 1.0498x; 1.0498x over previous
import functools

import jax
import jax.numpy as jnp
from jax import lax
from jax.experimental import pallas as pl
from jax.experimental.pallas import tpu as pltpu
from jax.experimental.pallas import tpu_sc as plsc

M = 1024
DH = 2048
E = 8
T = 4096
K = 2

BLK = 256
NB = (K * T) // BLK + E
P_PAD = NB * BLK

NW = 32

_SQRT_HALF = 0.7071067811865476


def _gating_body(x_ref, wg_ref, wgb_ref, wn_ref, wnb_ref, noise_ref,
                 idx_ref, gv_ref, imp_ref, load_ref, xbf_ref):
    t = pl.program_id(0)
    x = x_ref[...]
    xbf = x.astype(jnp.bfloat16)
    lo = lax.bitcast_convert_type(xbf[:, :M // 2], jnp.uint16)
    hi = lax.bitcast_convert_type(xbf[:, M // 2:], jnp.uint16)
    packed = lo.astype(jnp.uint32) | (hi.astype(jnp.uint32) << 16)
    xbf_ref[...] = lax.bitcast_convert_type(packed, jnp.int32)
    logits = lax.dot_general(
        x, wg_ref[...], (((1,), (1,)), ((), ())),
        preferred_element_type=jnp.float32) + wgb_ref[...]
    pre = lax.dot_general(
        x, wn_ref[...], (((1,), (1,)), ((), ())),
        preferred_element_type=jnp.float32) + wnb_ref[...]
    noise_scale = jax.nn.softplus(pre)
    h = logits + noise_ref[...] * noise_scale

    tm = h.shape[0]
    iota = lax.broadcasted_iota(jnp.int32, (tm, E), 1)
    neg_inf = jnp.float32(-jnp.inf)

    m1 = jnp.max(h, axis=1, keepdims=True)
    i1 = jnp.min(jnp.where(h == m1, iota, E), axis=1, keepdims=True)
    mask1 = iota == i1
    h2 = jnp.where(mask1, neg_inf, h)
    m2 = jnp.max(h2, axis=1, keepdims=True)
    i2 = jnp.min(jnp.where(h2 == m2, iota, E), axis=1, keepdims=True)
    mask2 = iota == i2
    h3 = jnp.where(mask2, neg_inf, h2)
    m3 = jnp.max(h3, axis=1, keepdims=True)

    e2 = jnp.exp(m2 - m1)
    denom = 1.0 + e2
    g1 = 1.0 / denom
    g2 = e2 / denom
    gates = jnp.where(mask1, g1, 0.0) + jnp.where(mask2, g2, 0.0)
    idx_ref[...] = jnp.concatenate([i1, i2], axis=1)
    gv_ref[...] = jnp.concatenate([g1, g2], axis=1)

    psi = jnp.where(h > m2, m2, jnp.where(h <= m3, m3, h))
    z = (logits - psi) / noise_scale
    p = 0.5 * (1.0 + lax.erf(z * _SQRT_HALF))

    imp_part = jnp.sum(gates, axis=0, keepdims=True)
    load_part = jnp.sum(p, axis=0, keepdims=True)

    @pl.when(t == 0)
    def _():
        imp_ref[...] = imp_part
        load_ref[...] = load_part

    @pl.when(t != 0)
    def _():
        imp_ref[...] += imp_part
        load_ref[...] += load_part


def _gating(flat, wg_w, wg_b, wn_w, wn_b, noise, tm=1024):
    nt = T // tm
    return pl.pallas_call(
        _gating_body,
        grid=(nt,),
        in_specs=[
            pl.BlockSpec((tm, M), lambda t: (t, 0)),
            pl.BlockSpec((E, M), lambda t: (0, 0)),
            pl.BlockSpec((1, E), lambda t: (0, 0)),
            pl.BlockSpec((E, M), lambda t: (0, 0)),
            pl.BlockSpec((1, E), lambda t: (0, 0)),
            pl.BlockSpec((tm, E), lambda t: (t, 0)),
        ],
        out_specs=[
            pl.BlockSpec((tm, 2), lambda t: (t, 0)),
            pl.BlockSpec((tm, 2), lambda t: (t, 0)),
            pl.BlockSpec((1, E), lambda t: (0, 0)),
            pl.BlockSpec((1, E), lambda t: (0, 0)),
            pl.BlockSpec((tm, M // 2), lambda t: (t, 0)),
        ],
        out_shape=[
            jax.ShapeDtypeStruct((T, 2), jnp.int32),
            jax.ShapeDtypeStruct((T, 2), jnp.float32),
            jax.ShapeDtypeStruct((1, E), jnp.float32),
            jax.ShapeDtypeStruct((1, E), jnp.float32),
            jax.ShapeDtypeStruct((T, M // 2), jnp.int32),
        ],
    )(flat, wg_w, wg_b, wn_w, wn_b, noise)


def _route(idx, gv):
    a = jnp.concatenate([idx[:, 0], idx[:, 1]])
    g = jnp.concatenate([gv[:, 0], gv[:, 1]])
    ar = jnp.arange(T, dtype=jnp.int32)
    tok = jnp.concatenate([ar, ar])
    onehot = (a[:, None] == jnp.arange(E, dtype=jnp.int32)[None, :])
    onehot = onehot.astype(jnp.int32)
    csum = jnp.cumsum(onehot, axis=0)
    counts = csum[-1]
    rank = jnp.sum(csum * onehot, axis=1) - 1
    cnt_pad = ((counts + BLK - 1) // BLK) * BLK
    ends = jnp.cumsum(cnt_pad)
    offs = ends - cnt_pad
    pos = jnp.sum(offs[None, :] * onehot, axis=1) + rank
    tok_of_slot = jnp.zeros((P_PAD,), jnp.int32).at[pos].set(tok)
    gate_of_slot = jnp.zeros((P_PAD,), jnp.float32).at[pos].set(g)
    bstart = jnp.arange(NB, dtype=jnp.int32) * BLK
    block_expert = jnp.minimum(
        jnp.sum((bstart[:, None] >= ends[None, :]).astype(jnp.int32), axis=1),
        E - 1).astype(jnp.int32)
    return tok_of_slot, gate_of_slot, block_expert, pos[:T], pos[T:]


def _sc_gather(flat_pk, tok_of_slot):
    m2 = M // 2
    n_slots = tok_of_slot.shape[0]
    rows_per_w = n_slots // NW
    c = 80
    nch = rows_per_w // c
    assert rows_per_w % c == 0
    mesh = plsc.VectorSubcoreMesh(core_axis_name="c", subcore_axis_name="s")

    @functools.partial(
        pl.kernel, mesh=mesh,
        out_type=jax.ShapeDtypeStruct((n_slots, m2), jnp.int32),
        scratch_types=[
            pltpu.VMEM((rows_per_w,), jnp.int32),
            pltpu.VMEM((c, m2), jnp.int32),
            pltpu.VMEM((c, m2), jnp.int32),
            pltpu.SemaphoreType.DMA,
            pltpu.SemaphoreType.DMA,
        ],
    )
    def k(flat_hbm, idx_hbm, out_hbm, idx_v, buf_a, buf_b, sem_a, sem_b):
        wid = lax.axis_index("s") * 2 + lax.axis_index("c")
        base = wid * rows_per_w
        pltpu.sync_copy(idx_hbm.at[pl.ds(base, rows_per_w)], idx_v)
        bufs = (buf_a, buf_b)
        sems = (sem_a, sem_b)
        cps = [None, None]
        for ch in range(nch):
            p = ch % 2
            cps[p] = pltpu.async_copy(
                flat_hbm.at[idx_v.at[pl.ds(ch * c, c)]], bufs[p], sems[p])
            if ch >= 1:
                q = (ch - 1) % 2
                cps[q].wait()
                pltpu.sync_copy(bufs[q],
                                out_hbm.at[pl.ds(base + (ch - 1) * c, c)])
        cps[(nch - 1) % 2].wait()
        pltpu.sync_copy(bufs[(nch - 1) % 2],
                        out_hbm.at[pl.ds(base + (nch - 1) * c, c)])

    return k(flat_pk, tok_of_slot)


def _expert_body(be_ref, x_ref, g_ref, ww_ref, wb_ref, vw_ref, vb_ref,
                 w2w_ref, w2b_ref, *rest):
    out_ref = rest[-1]
    u = lax.bitcast_convert_type(x_ref[...], jnp.uint32)
    lo = lax.bitcast_convert_type((u & 0xFFFF).astype(jnp.uint16),
                                  jnp.bfloat16)
    hi = lax.bitcast_convert_type((u >> 16).astype(jnp.uint16),
                                  jnp.bfloat16)
    x = jnp.concatenate([lo, hi], axis=1)
    xv = lax.dot_general(
        x, vw_ref[...], (((1,), (1,)), ((), ())),
        preferred_element_type=jnp.float32) + vb_ref[...]
    xw = lax.dot_general(
        x, ww_ref[0], (((1,), (1,)), ((), ())),
        preferred_element_type=jnp.float32) + wb_ref[0]
    act = xw * jax.nn.sigmoid(xw) * xv
    contrib = lax.dot_general(
        act.astype(jnp.bfloat16), w2w_ref[...], (((1,), (1,)), ((), ())),
        preferred_element_type=jnp.float32) + w2b_ref[...]
    g = g_ref[0, 0, :].reshape(-1, 1)
    out_ref[...] = g * contrib


def _experts_half(be, x_half, g_half, ww, wb, vw, vb, w2w, w2b,
                  prev=None, blk_off=0):
    nb2 = be.shape[0]
    in_specs = [
        pl.BlockSpec((BLK, M // 2), lambda b, e: (b, 0)),
        pl.BlockSpec((1, 1, BLK), lambda b, e: (b, 0, 0)),
        pl.BlockSpec((1, DH, M), lambda b, e: (e[b], 0, 0)),
        pl.BlockSpec((1, 1, DH), lambda b, e: (e[b], 0, 0)),
        pl.BlockSpec((DH, M), lambda b, e: (0, 0)),
        pl.BlockSpec((1, DH), lambda b, e: (0, 0)),
        pl.BlockSpec((M, DH), lambda b, e: (0, 0)),
        pl.BlockSpec((1, M), lambda b, e: (0, 0)),
    ]
    args = [be, x_half, g_half.reshape(nb2, 1, BLK), ww, wb, vw, vb,
            w2w, w2b]
    aliases = {}
    if prev is not None:
        in_specs.append(pl.BlockSpec(memory_space=pl.ANY))
        args.append(prev)
        aliases = {9: 0}
    grid_spec = pltpu.PrefetchScalarGridSpec(
        num_scalar_prefetch=1,
        grid=(nb2,),
        in_specs=in_specs,
        out_specs=pl.BlockSpec((BLK, M), lambda b, e: (b + blk_off, 0)),
    )
    return pl.pallas_call(
        _expert_body,
        grid_spec=grid_spec,
        out_shape=jax.ShapeDtypeStruct((P_PAD, M), jnp.float32),
        input_output_aliases=aliases,
    )(*args)


def _sc_combine(rows, slot1, slot2):
    rows_per_w = T // NW
    c = 32
    nch = rows_per_w // c
    nsl = M // 16
    mesh = plsc.VectorSubcoreMesh(core_axis_name="c", subcore_axis_name="s")

    @functools.partial(
        pl.kernel, mesh=mesh,
        out_type=jax.ShapeDtypeStruct((T, M), jnp.float32),
        scratch_types=[
            pltpu.VMEM((c,), jnp.int32),
            pltpu.VMEM((c,), jnp.int32),
            pltpu.VMEM((c, M), jnp.float32),
            pltpu.VMEM((c, M), jnp.float32),
            pltpu.SemaphoreType.DMA,
            pltpu.SemaphoreType.DMA,
        ],
    )
    def k(rows_hbm, s1_hbm, s2_hbm, out_hbm, i1_v, i2_v, a_v, b_v, sa, sb):
        wid = lax.axis_index("s") * 2 + lax.axis_index("c")
        base = wid * rows_per_w

        def body(i, carry):
            b = base + i * c
            pltpu.sync_copy(s1_hbm.at[pl.ds(b, c)], i1_v)
            pltpu.sync_copy(s2_hbm.at[pl.ds(b, c)], i2_v)
            cp1 = pltpu.async_copy(rows_hbm.at[i1_v], a_v, sa)
            cp2 = pltpu.async_copy(rows_hbm.at[i2_v], b_v, sb)
            cp1.wait()
            cp2.wait()

            def add_row(j, carry2):
                for s in range(nsl):
                    sl = pl.ds(s * 16, 16)
                    a_v[j, sl] = a_v[j, sl] + b_v[j, sl]
                return carry2

            lax.fori_loop(0, c, add_row, 0)
            pltpu.sync_copy(a_v, out_hbm.at[pl.ds(b, c)])
            return carry

        lax.fori_loop(0, nch, body, 0)

    return k(rows, slot1, slot2)


def _cv_loss(v):
    return 0.01 * jnp.std(v) / (jnp.mean(v) + 1e-6)


def kernel(x, Wg_w, Wg_b, Wn_w, Wn_b, W_w, W_b, V_w, V_b, W2_w, W2_b):
    B, N, m = x.shape
    flat = x.reshape(B * N, m)
    noise = jax.random.normal(jax.random.key(1234), (T, E),
                              dtype=jnp.float32)

    idx, gv, imp, load, flat_pk = _gating(
        flat, Wg_w, Wg_b.reshape(1, E), Wn_w, Wn_b.reshape(1, E), noise)
    l_moe = _cv_loss(imp[0]) + _cv_loss(load[0])

    tok_of_slot, gate_of_slot, block_expert, slot1, slot2 = _route(idx, gv)

    half = P_PAD // 2
    nb2 = NB // 2
    x_a = _sc_gather(flat_pk, tok_of_slot[:half])
    x_b = _sc_gather(flat_pk, tok_of_slot[half:])

    wts = (W_w.astype(jnp.bfloat16), W_b.reshape(E, 1, DH),
           V_w.astype(jnp.bfloat16), V_b.reshape(1, DH),
           W2_w.astype(jnp.bfloat16), W2_b.reshape(1, M))
    rows_a = _experts_half(block_expert[:nb2], x_a, gate_of_slot[:half],
                           *wts)
    rows = _experts_half(block_expert[nb2:], x_b, gate_of_slot[half:],
                         *wts, prev=rows_a, blk_off=nb2)

    flat_out = _sc_combine(rows, slot1, slot2)
    return (flat_out.reshape(B, N, m), l_moe)
